# Initial kernel scaffold; baseline (speedup 1.0000x reference)
import functools

import jax
import jax.numpy as jnp
from jax import lax
from jax.experimental import pallas as pl
from jax.experimental.pallas import tpu as pltpu
from jax.experimental.pallas import tpu_sc as plsc

NC = 2
NS = 16
NW = NC * NS
W = 128
NBUF = 4


@functools.lru_cache(maxsize=None)
def _make_lookup(B, D):
    b_per_w = B // NW
    nchunk = b_per_w // W
    ngroup = nchunk // NBUF
    assert b_per_w * NW == B and W * nchunk == b_per_w and NBUF * ngroup == nchunk

    mesh = plsc.VectorSubcoreMesh(core_axis_name="c", subcore_axis_name="s")

    @functools.partial(
        pl.kernel,
        mesh=mesh,
        out_type=(
            jax.ShapeDtypeStruct((B, D), jnp.float32),
            jax.ShapeDtypeStruct((B, D), jnp.float32),
        ),
        scratch_types=(
            [pltpu.VMEM((b_per_w,), jnp.int32)] * 2
            + [pltpu.VMEM((W, D), jnp.float32)] * NBUF
            + [pltpu.SemaphoreType.DMA] * (2 * NBUF)
        ),
    )
    def lookup(tok_idx_hbm, pos_idx_hbm, tok_tab, pos_tab, tok_out, pos_out,
               tok_idx_v, pos_idx_v, *scratch):
        rows = scratch[:NBUF]
        gsems = scratch[NBUF:2 * NBUF]
        osems = scratch[2 * NBUF:]

        wid = lax.axis_index("s") * NC + lax.axis_index("c")
        base = pl.multiple_of(wid * b_per_w, 8)

        pltpu.sync_copy(tok_idx_hbm.at[pl.ds(base, b_per_w)], tok_idx_v)
        pltpu.sync_copy(pos_idx_hbm.at[pl.ds(base, b_per_w)], pos_idx_v)

        def run_table(tab, idx_v, out):
            def gdesc(c, b):
                start = pl.multiple_of(c * W, 8)
                return pltpu.make_async_copy(
                    tab.at[idx_v.at[pl.ds(start, W)]], rows[b], gsems[b])

            def odesc(c, b):
                start = pl.multiple_of(base + c * W, 8)
                return pltpu.make_async_copy(
                    rows[b], out.at[pl.ds(start, W)], osems[b])

            for b in range(NBUF):
                gdesc(b, b).start()

            def body(j, carry):
                for b in range(NBUF):
                    c = j * NBUF + b
                    gdesc(c, b).wait()
                    odesc(c, b).start()
                for b in range(NBUF):
                    c = j * NBUF + b
                    odesc(c, b).wait()
                    gdesc(c + NBUF, b).start()
                return carry

            lax.fori_loop(0, ngroup - 1, body, 0)

            last = (ngroup - 1) * NBUF
            for b in range(NBUF):
                gdesc(last + b, b).wait()
                odesc(last + b, b).start()
            for b in range(NBUF):
                odesc(last + b, b).wait()

        run_table(tok_tab, tok_idx_v, tok_out)
        run_table(pos_tab, pos_idx_v, pos_out)

    return lookup


def kernel(tokens, pos, token_table, pos_table):
    S0, S1 = tokens.shape
    B = S0 * S1
    D = token_table.shape[1]
    tok_flat = tokens.reshape(B).astype(jnp.int32)
    pos_flat = pos.reshape(B).astype(jnp.int32)
    tok_out, pos_out = _make_lookup(B, D)(
        tok_flat, pos_flat, token_table, pos_table)
    return tok_out.reshape(S0, S1, D), pos_out.reshape(S0, S1, D)


# speedup vs baseline: 1.9015x; 1.9015x over previous
import functools

import jax
import jax.numpy as jnp
from jax import lax
from jax.experimental import pallas as pl
from jax.experimental.pallas import tpu as pltpu
from jax.experimental.pallas import tpu_sc as plsc

NC = 2
NS = 16
NW = NC * NS
W = 128
NBUF = 4


@functools.lru_cache(maxsize=None)
def _make_lookup(B, D):
    b_per_w = B // NW
    nchunk = b_per_w // W
    ngroup = nchunk // NBUF
    assert b_per_w * NW == B and W * nchunk == b_per_w and NBUF * ngroup == nchunk

    mesh = plsc.VectorSubcoreMesh(core_axis_name="c", subcore_axis_name="s")

    @functools.partial(
        pl.kernel,
        mesh=mesh,
        compiler_params=pltpu.CompilerParams(use_tc_tiling_on_sc=False),
        out_type=(
            jax.ShapeDtypeStruct((B, D), jnp.float32),
            jax.ShapeDtypeStruct((B, D), jnp.float32),
        ),
        scratch_types=(
            [pltpu.VMEM((b_per_w,), jnp.int32)] * 2
            + [pltpu.VMEM((W, D), jnp.float32)] * NBUF
            + [pltpu.SemaphoreType.DMA] * (2 * NBUF)
        ),
    )
    def lookup(tok_idx_hbm, pos_idx_hbm, tok_tab, pos_tab, tok_out, pos_out,
               tok_idx_v, pos_idx_v, *scratch):
        rows = scratch[:NBUF]
        gsems = scratch[NBUF:2 * NBUF]
        osems = scratch[2 * NBUF:]

        wid = lax.axis_index("s") * NC + lax.axis_index("c")
        base = pl.multiple_of(wid * b_per_w, 8)

        pltpu.sync_copy(tok_idx_hbm.at[pl.ds(base, b_per_w)], tok_idx_v)
        pltpu.sync_copy(pos_idx_hbm.at[pl.ds(base, b_per_w)], pos_idx_v)

        def run_table(tab, idx_v, out):
            def gdesc(c, b):
                start = pl.multiple_of(c * W, 8)
                return pltpu.make_async_copy(
                    tab.at[idx_v.at[pl.ds(start, W)]], rows[b], gsems[b])

            def odesc(c, b):
                start = pl.multiple_of(base + c * W, 8)
                return pltpu.make_async_copy(
                    rows[b], out.at[pl.ds(start, W)], osems[b])

            for b in range(NBUF):
                gdesc(b, b).start()

            def body(j, carry):
                for b in range(NBUF):
                    c = j * NBUF + b
                    gdesc(c, b).wait()
                    odesc(c, b).start()
                for b in range(NBUF):
                    c = j * NBUF + b
                    odesc(c, b).wait()
                    gdesc(c + NBUF, b).start()
                return carry

            lax.fori_loop(0, ngroup - 1, body, 0)

            last = (ngroup - 1) * NBUF
            for b in range(NBUF):
                gdesc(last + b, b).wait()
                odesc(last + b, b).start()
            for b in range(NBUF):
                odesc(last + b, b).wait()

        run_table(tok_tab, tok_idx_v, tok_out)
        run_table(pos_tab, pos_idx_v, pos_out)

    return lookup


def kernel(tokens, pos, token_table, pos_table):
    S0, S1 = tokens.shape
    B = S0 * S1
    D = token_table.shape[1]
    tok_flat = tokens.reshape(B).astype(jnp.int32)
    pos_flat = pos.reshape(B).astype(jnp.int32)
    tok_out, pos_out = _make_lookup(B, D)(
        tok_flat, pos_flat, token_table, pos_table)
    return tok_out.reshape(S0, S1, D), pos_out.reshape(S0, S1, D)
